# Initial kernel scaffold; baseline (speedup 1.0000x reference)
import jax
import jax.numpy as jnp
from jax.experimental import pallas as pl


def kernel(x, router_w, router_b, w1, b1, w2, b2):
    raise NotImplementedError("write your pallas kernel here")


# speedup vs baseline: 3.6855x; 3.6855x over previous
import functools
import math

import jax
import jax.numpy as jnp
from jax import lax
from jax.experimental import pallas as pl
from jax.experimental.pallas import tpu as pltpu
from jax.experimental.pallas import tpu_sc as plsc

DIM = 1024
HIDDEN = 512
E = 64
N = 2048
CAP = 36
CAPP = 40
NSLOT = E * CAPP
AUX_W = 0.01
NWORK = 32


def _router_body(probs_ref, eidx_ref, gate_ref, stok_ref, gslot_ref,
                 sgat_ref, aux_ref):
    probs = probs_ref[...]
    eidx = eidx_ref[...]
    gate = gate_ref[...]

    iota_e = lax.broadcasted_iota(jnp.int32, (N, E), 1)
    oh = (iota_e == eidx).astype(jnp.float32)
    counts = jnp.sum(oh, axis=0, keepdims=True)
    rppe = jnp.sum(probs, axis=0, keepdims=True) * (1.0 / N)
    aux = jnp.sum((counts * (1.0 / N)) * rppe, axis=1, keepdims=True)
    aux_ref[...] = aux * (E * AUX_W)
    iota_e1 = lax.broadcasted_iota(jnp.int32, (1, E), 1)
    cmin = jnp.min(counts, axis=1, keepdims=True)
    emin = jnp.min(jnp.where(counts == cmin, iota_e1, E), axis=1,
                   keepdims=True)
    empty_slot = emin * CAPP + (CAPP - 1)

    tid = lax.broadcasted_iota(jnp.int32, (N, 1), 0)
    eidx_t = jnp.reshape(eidx, (1, N))
    gate_t = jnp.reshape(gate, (1, N))
    tid_t = jnp.reshape(tid, (1, N))
    chunks = []
    ck = 256
    for c in range(N // ck):
        eT = eidx_t[:, c * ck:(c + 1) * ck]
        gT = gate_t[:, c * ck:(c + 1) * ck]
        tT = tid_t[:, c * ck:(c + 1) * ck]
        beats = (eidx == eT) & ((gate > gT) | ((gate == gT) & (tid < tT)))
        chunks.append(jnp.sum(beats.astype(jnp.float32), axis=0,
                              keepdims=True))
    rank_t = jnp.concatenate(chunks, axis=1).astype(jnp.int32)

    kept_t = rank_t < CAP
    slot_t = eidx_t * CAPP + rank_t
    slot_full_t = jnp.where(kept_t, slot_t, NSLOT)
    sgat_t = jnp.where(kept_t, slot_t, empty_slot)
    sgat_ref[...] = jnp.reshape(sgat_t, (N // 128, 128))

    tid_tf = tid_t.astype(jnp.float32)
    for c in range(NSLOT // ck):
        s_ids = lax.broadcasted_iota(jnp.int32, (ck, 1), 0) + c * ck
        msel = (slot_full_t == s_ids).astype(jnp.float32)
        stok = jnp.sum(msel * tid_tf, axis=1, keepdims=True)
        gsl = jnp.sum(msel * gate_t, axis=1, keepdims=True)
        rows = ck // 128
        stok_ref[c * rows:(c + 1) * rows, :] = (
            jnp.reshape(stok, (rows, 128)).astype(jnp.int32))
        gslot_ref[c * rows:(c + 1) * rows, :] = jnp.reshape(gsl, (rows, 128))


def _router_call(probs, eidx2d, gate2d):
    return pl.pallas_call(
        _router_body,
        out_shape=(
            jax.ShapeDtypeStruct((NSLOT // 128, 128), jnp.int32),
            jax.ShapeDtypeStruct((NSLOT // 128, 128), jnp.float32),
            jax.ShapeDtypeStruct((N // 128, 128), jnp.int32),
            jax.ShapeDtypeStruct((1, 1), jnp.float32),
        ),
    )(probs, eidx2d, gate2d)


def _expert_body(xd_ref, w1_ref, b1_ref, w2_ref, b2_ref, g_ref, y_ref):
    xg = xd_ref[...]
    h = lax.dot_general(xg, w1_ref[0], (((1,), (1,)), ((), ())),
                        preferred_element_type=jnp.float32)
    h = h + b1_ref[0]
    h = h * 0.5 * (1.0 + lax.erf(h * (1.0 / math.sqrt(2.0))))
    y = lax.dot_general(h, w2_ref[0], (((1,), (1,)), ((), ())),
                        preferred_element_type=jnp.float32)
    y = y + b2_ref[0]
    g = jnp.reshape(g_ref[...], (CAPP, 1))
    y_ref[...] = y * g


def _expert_call(xd, w1, b1r, w2, b2r, gslot):
    return pl.pallas_call(
        _expert_body,
        grid=(E,),
        out_shape=jax.ShapeDtypeStruct((NSLOT, DIM), jnp.float32),
        in_specs=[
            pl.BlockSpec((CAPP, DIM), lambda e: (e, 0)),
            pl.BlockSpec((1, HIDDEN, DIM), lambda e: (e, 0, 0)),
            pl.BlockSpec((1, 1, HIDDEN), lambda e: (e, 0, 0)),
            pl.BlockSpec((1, DIM, HIDDEN), lambda e: (e, 0, 0)),
            pl.BlockSpec((1, 1, DIM), lambda e: (e, 0, 0)),
            pl.BlockSpec((1, 1, CAPP), lambda e: (e, 0, 0)),
        ],
        out_specs=pl.BlockSpec((CAPP, DIM), lambda e: (e, 0)),
    )(xd, w1, b1r, w2, b2r, gslot)


@functools.lru_cache(maxsize=None)
def _make_sc_gather(n_rows, n_cols):
    rows_per = n_rows // NWORK
    mesh = plsc.VectorSubcoreMesh(core_axis_name="c", subcore_axis_name="s",
                                  num_cores=2, num_subcores=16)

    @functools.partial(
        pl.kernel,
        out_type=jax.ShapeDtypeStruct((n_rows, n_cols), jnp.float32),
        mesh=mesh,
        scratch_types=[
            pltpu.VMEM((rows_per,), jnp.int32),
            pltpu.VMEM((rows_per, n_cols), jnp.float32),
            pltpu.SemaphoreType.DMA,
        ],
    )
    def gather(table_hbm, idx_hbm, out_hbm, idx_v, rows_v, sem):
        wid = lax.axis_index("s") * 2 + lax.axis_index("c")
        base = wid * rows_per
        pltpu.sync_copy(idx_hbm.at[pl.ds(base, rows_per)], idx_v)
        pltpu.async_copy(table_hbm.at[idx_v], rows_v, sem).wait()
        pltpu.sync_copy(rows_v, out_hbm.at[pl.ds(base, rows_per)])

    return gather


def kernel(x, router_w, router_b, w1, b1, w2, b2):
    b, n, d = x.shape
    flat = x.reshape(n, d)
    logits = flat @ router_w.T + router_b
    probs = jax.nn.softmax(logits.astype(jnp.float32), axis=-1)
    topk_vals, topk_idx = lax.top_k(probs, 1)
    topk_vals = topk_vals / (jnp.sum(topk_vals, axis=-1, keepdims=True)
                             + 1e-9)
    eidx2d = topk_idx.astype(jnp.int32)
    gate2d = topk_vals
    stok2d, gslot2d, sgat2d, aux2d = _router_call(probs, eidx2d, gate2d)
    stok = stok2d.reshape(NSLOT)
    sgat = sgat2d.reshape(N)
    gslot = gslot2d.reshape(E, 1, CAPP)
    xd = _make_sc_gather(NSLOT, DIM)(flat, stok)
    yd = _expert_call(xd, w1, b1.reshape(E, 1, HIDDEN), w2,
                      b2.reshape(E, 1, DIM), gslot)
    out = _make_sc_gather(N, DIM)(yd, sgat)
    return out.reshape(b, n, d), aux2d[0, 0]


# speedup vs baseline: 6.0010x; 1.6283x over previous
import functools
import math

import jax
import jax.numpy as jnp
from jax import lax
from jax.experimental import pallas as pl
from jax.experimental.pallas import tpu as pltpu
from jax.experimental.pallas import tpu_sc as plsc

DIM = 1024
HIDDEN = 512
E = 64
N = 2048
CAP = 36
CAPP = 40
NSLOT = E * CAPP
AUX_W = 0.01
NWORK = 32


def _router_body(probs_ref, eidx_ref, gate_ref, stok_ref, gslot_ref,
                 sgat_ref, aux_ref):
    probs = probs_ref[...]
    eidx = eidx_ref[...]
    gate = gate_ref[...]

    iota_e = lax.broadcasted_iota(jnp.int32, (N, E), 1)
    oh = (iota_e == eidx).astype(jnp.float32)
    counts = jnp.sum(oh, axis=0, keepdims=True)
    rppe = jnp.sum(probs, axis=0, keepdims=True) * (1.0 / N)
    aux = jnp.sum((counts * (1.0 / N)) * rppe, axis=1, keepdims=True)
    aux_ref[...] = aux * (E * AUX_W)
    iota_e1 = lax.broadcasted_iota(jnp.int32, (1, E), 1)
    cmin = jnp.min(counts, axis=1, keepdims=True)
    emin = jnp.min(jnp.where(counts == cmin, iota_e1, E), axis=1,
                   keepdims=True)
    empty_slot = emin * CAPP + (CAPP - 1)

    tid = lax.broadcasted_iota(jnp.int32, (N, 1), 0)
    eidx_t = jnp.reshape(eidx, (1, N))
    gate_t = jnp.reshape(gate, (1, N))
    tid_t = jnp.reshape(tid, (1, N))
    chunks = []
    ck = 256
    for c in range(N // ck):
        eT = eidx_t[:, c * ck:(c + 1) * ck]
        gT = gate_t[:, c * ck:(c + 1) * ck]
        tT = tid_t[:, c * ck:(c + 1) * ck]
        beats = (eidx == eT) & ((gate > gT) | ((gate == gT) & (tid < tT)))
        chunks.append(jnp.sum(beats.astype(jnp.float32), axis=0,
                              keepdims=True))
    rank_t = jnp.concatenate(chunks, axis=1).astype(jnp.int32)

    kept_t = rank_t < CAP
    slot_t = eidx_t * CAPP + rank_t
    slot_full_t = jnp.where(kept_t, slot_t, NSLOT)
    sgat_t = jnp.where(kept_t, slot_t, empty_slot)
    sgat_ref[...] = jnp.reshape(sgat_t, (N // 128, 128))

    tid_tf = tid_t.astype(jnp.float32)
    for c in range(NSLOT // ck):
        s_ids = lax.broadcasted_iota(jnp.int32, (ck, 1), 0) + c * ck
        msel = (slot_full_t == s_ids).astype(jnp.float32)
        stok = jnp.sum(msel * tid_tf, axis=1, keepdims=True)
        gsl = jnp.sum(msel * gate_t, axis=1, keepdims=True)
        rows = ck // 128
        stok_ref[c * rows:(c + 1) * rows, :] = (
            jnp.reshape(stok, (rows, 128)).astype(jnp.int32))
        gslot_ref[c * rows:(c + 1) * rows, :] = jnp.reshape(gsl, (rows, 128))


def _router_call(probs, eidx2d, gate2d):
    return pl.pallas_call(
        _router_body,
        out_shape=(
            jax.ShapeDtypeStruct((NSLOT // 128, 128), jnp.int32),
            jax.ShapeDtypeStruct((NSLOT // 128, 128), jnp.float32),
            jax.ShapeDtypeStruct((N // 128, 128), jnp.int32),
            jax.ShapeDtypeStruct((1, 1), jnp.float32),
        ),
    )(probs, eidx2d, gate2d)


def _expert_body(xd_ref, w1_ref, b1_ref, w2_ref, b2_ref, g_ref, y_ref):
    xg = xd_ref[...]
    h = lax.dot_general(xg, w1_ref[0], (((1,), (1,)), ((), ())),
                        preferred_element_type=jnp.float32)
    h = h + b1_ref[0]
    h = h * 0.5 * (1.0 + lax.erf(h * (1.0 / math.sqrt(2.0))))
    y = lax.dot_general(h, w2_ref[0], (((1,), (1,)), ((), ())),
                        preferred_element_type=jnp.float32)
    y = y + b2_ref[0]
    g = jnp.reshape(g_ref[...], (CAPP, 1))
    y_ref[...] = y * g


def _expert_call(xd, w1, b1r, w2, b2r, gslot):
    return pl.pallas_call(
        _expert_body,
        grid=(E,),
        out_shape=jax.ShapeDtypeStruct((NSLOT, DIM), jnp.float32),
        in_specs=[
            pl.BlockSpec((CAPP, DIM), lambda e: (e, 0)),
            pl.BlockSpec((1, HIDDEN, DIM), lambda e: (e, 0, 0)),
            pl.BlockSpec((1, 1, HIDDEN), lambda e: (e, 0, 0)),
            pl.BlockSpec((1, DIM, HIDDEN), lambda e: (e, 0, 0)),
            pl.BlockSpec((1, 1, DIM), lambda e: (e, 0, 0)),
            pl.BlockSpec((1, 1, CAPP), lambda e: (e, 0, 0)),
        ],
        out_specs=pl.BlockSpec((CAPP, DIM), lambda e: (e, 0)),
    )(xd, w1, b1r, w2, b2r, gslot)


@functools.lru_cache(maxsize=None)
def _make_sc_gather(n_rows, n_cols):
    rows_per = n_rows // NWORK
    mesh = plsc.VectorSubcoreMesh(core_axis_name="c", subcore_axis_name="s",
                                  num_cores=2, num_subcores=16)

    @functools.partial(
        pl.kernel,
        out_type=jax.ShapeDtypeStruct((n_rows, n_cols), jnp.float32),
        mesh=mesh,
        scratch_types=[
            pltpu.VMEM((rows_per,), jnp.int32),
            pltpu.VMEM((rows_per, n_cols), jnp.float32),
            pltpu.SemaphoreType.DMA,
        ],
    )
    def gather(table_hbm, idx_hbm, out_hbm, idx_v, rows_v, sem):
        wid = lax.axis_index("s") * 2 + lax.axis_index("c")
        base = wid * rows_per
        pltpu.sync_copy(idx_hbm.at[pl.ds(base, rows_per)], idx_v)
        pltpu.async_copy(table_hbm.at[idx_v], rows_v, sem).wait()
        pltpu.sync_copy(rows_v, out_hbm.at[pl.ds(base, rows_per)])

    return gather


def kernel(x, router_w, router_b, w1, b1, w2, b2):
    b, n, d = x.shape
    flat = x.reshape(n, d)
    logits = flat @ router_w.T + router_b
    probs = jax.nn.softmax(logits.astype(jnp.float32), axis=-1)
    pmax = jnp.max(probs, axis=-1, keepdims=True)
    eidx2d = jnp.argmax(probs, axis=-1, keepdims=True).astype(jnp.int32)
    gate2d = pmax / (pmax + 1e-9)
    stok2d, gslot2d, sgat2d, aux2d = _router_call(probs, eidx2d, gate2d)
    stok = stok2d.reshape(NSLOT)
    sgat = sgat2d.reshape(N)
    gslot = gslot2d.reshape(E, 1, CAPP)
    xd = _make_sc_gather(NSLOT, DIM)(flat, stok)
    yd = _expert_call(xd, w1, b1.reshape(E, 1, HIDDEN), w2,
                      b2.reshape(E, 1, DIM), gslot)
    out = _make_sc_gather(N, DIM)(yd, sgat)
    return out.reshape(b, n, d), aux2d[0, 0]


# speedup vs baseline: 7.8956x; 1.3157x over previous
import functools
import math

import jax
import jax.numpy as jnp
from jax import lax
from jax.experimental import pallas as pl
from jax.experimental.pallas import tpu as pltpu
from jax.experimental.pallas import tpu_sc as plsc

DIM = 1024
HIDDEN = 512
E = 64
N = 2048
CAP = 36
CAPP = 40
NSLOT = E * CAPP
AUX_W = 0.01
NWORK = 32


def _router_body(probs_ref, eidx_ref, gate_ref, stok_ref, gslot_ref,
                 sgat_ref, aux_ref):
    probs = probs_ref[...]
    eidx = eidx_ref[...]
    gate = gate_ref[...]

    iota_e = lax.broadcasted_iota(jnp.int32, (N, E), 1)
    oh = (iota_e == eidx).astype(jnp.float32)
    counts = jnp.sum(oh, axis=0, keepdims=True)
    rppe = jnp.sum(probs, axis=0, keepdims=True) * (1.0 / N)
    aux = jnp.sum((counts * (1.0 / N)) * rppe, axis=1, keepdims=True)
    aux_ref[...] = aux * (E * AUX_W)
    iota_e1 = lax.broadcasted_iota(jnp.int32, (1, E), 1)
    cmin = jnp.min(counts, axis=1, keepdims=True)
    emin = jnp.min(jnp.where(counts == cmin, iota_e1, E), axis=1,
                   keepdims=True)
    empty_slot = emin * CAPP + (CAPP - 1)

    tid = lax.broadcasted_iota(jnp.int32, (N, 1), 0)
    eidx_t = jnp.reshape(eidx, (1, N))
    gate_t = jnp.reshape(gate, (1, N))
    tid_t = jnp.reshape(tid, (1, N))
    chunks = []
    ck = 256
    for c in range(N // ck):
        eT = eidx_t[:, c * ck:(c + 1) * ck]
        gT = gate_t[:, c * ck:(c + 1) * ck]
        tT = tid_t[:, c * ck:(c + 1) * ck]
        beats = (eidx == eT) & ((gate > gT) | ((gate == gT) & (tid < tT)))
        chunks.append(jnp.sum(beats.astype(jnp.float32), axis=0,
                              keepdims=True))
    rank_t = jnp.concatenate(chunks, axis=1).astype(jnp.int32)

    kept_t = rank_t < CAP
    slot_t = eidx_t * CAPP + rank_t
    slot_full_t = jnp.where(kept_t, slot_t, NSLOT)
    sgat_t = jnp.where(kept_t, slot_t, empty_slot)
    sgat_ref[...] = jnp.reshape(sgat_t, (N // 128, 128))

    tid_tf = tid_t.astype(jnp.float32)
    for c in range(NSLOT // ck):
        s_ids = lax.broadcasted_iota(jnp.int32, (ck, 1), 0) + c * ck
        msel = (slot_full_t == s_ids).astype(jnp.float32)
        stok = jnp.sum(msel * tid_tf, axis=1, keepdims=True)
        gsl = jnp.sum(msel * gate_t, axis=1, keepdims=True)
        m_any = jnp.sum(msel, axis=1, keepdims=True)
        stok_i = jnp.where(m_any > 0.0, stok.astype(jnp.int32),
                           lax.rem(s_ids, N))
        rows = ck // 128
        stok_ref[c * rows:(c + 1) * rows, :] = jnp.reshape(stok_i,
                                                           (rows, 128))
        gslot_ref[c * rows:(c + 1) * rows, :] = jnp.reshape(gsl, (rows, 128))


def _router_call(probs, eidx2d, gate2d):
    return pl.pallas_call(
        _router_body,
        out_shape=(
            jax.ShapeDtypeStruct((NSLOT // 128, 128), jnp.int32),
            jax.ShapeDtypeStruct((NSLOT // 128, 128), jnp.float32),
            jax.ShapeDtypeStruct((N // 128, 128), jnp.int32),
            jax.ShapeDtypeStruct((1, 1), jnp.float32),
        ),
    )(probs, eidx2d, gate2d)


EG = 2


def _expert_body(xd_ref, w1_ref, b1_ref, w2_ref, b2_ref, g_ref, y_ref):
    xg = xd_ref[...].reshape(EG, CAPP, DIM)
    h = lax.dot_general(xg, w1_ref[...], (((2,), (2,)), ((0,), (0,))),
                        preferred_element_type=jnp.float32)
    h = h + b1_ref[...]
    h = h * 0.5 * (1.0 + lax.erf(h * (1.0 / math.sqrt(2.0))))
    y = lax.dot_general(h, w2_ref[...], (((2,), (2,)), ((0,), (0,))),
                        preferred_element_type=jnp.float32)
    y = y + b2_ref[...]
    g = jnp.transpose(g_ref[...], (0, 2, 1))
    y_ref[...] = (y * g).reshape(EG * CAPP, DIM)


def _expert_call(xd, w1, b1r, w2, b2r, gslot):
    return pl.pallas_call(
        _expert_body,
        grid=(E // EG,),
        out_shape=jax.ShapeDtypeStruct((NSLOT, DIM), jnp.float32),
        in_specs=[
            pl.BlockSpec((EG * CAPP, DIM), lambda e: (e, 0)),
            pl.BlockSpec((EG, HIDDEN, DIM), lambda e: (e, 0, 0)),
            pl.BlockSpec((EG, 1, HIDDEN), lambda e: (e, 0, 0)),
            pl.BlockSpec((EG, DIM, HIDDEN), lambda e: (e, 0, 0)),
            pl.BlockSpec((EG, 1, DIM), lambda e: (e, 0, 0)),
            pl.BlockSpec((EG, 1, CAPP), lambda e: (e, 0, 0)),
        ],
        out_specs=pl.BlockSpec((EG * CAPP, DIM), lambda e: (e, 0)),
    )(xd, w1, b1r, w2, b2r, gslot)


@functools.lru_cache(maxsize=None)
def _make_sc_gather(n_rows, n_cols):
    rows_per = n_rows // NWORK
    mesh = plsc.VectorSubcoreMesh(core_axis_name="c", subcore_axis_name="s",
                                  num_cores=2, num_subcores=16)

    @functools.partial(
        pl.kernel,
        out_type=jax.ShapeDtypeStruct((n_rows, n_cols), jnp.float32),
        mesh=mesh,
        scratch_types=[
            pltpu.VMEM((rows_per,), jnp.int32),
            pltpu.VMEM((rows_per, n_cols), jnp.float32),
            pltpu.SemaphoreType.DMA,
        ],
    )
    def gather(table_hbm, idx_hbm, out_hbm, idx_v, rows_v, sem):
        wid = lax.axis_index("s") * 2 + lax.axis_index("c")
        base = wid * rows_per
        pltpu.sync_copy(idx_hbm.at[pl.ds(base, rows_per)], idx_v)
        pltpu.async_copy(table_hbm.at[idx_v], rows_v, sem).wait()
        pltpu.sync_copy(rows_v, out_hbm.at[pl.ds(base, rows_per)])

    return gather


def kernel(x, router_w, router_b, w1, b1, w2, b2):
    b, n, d = x.shape
    flat = x.reshape(n, d)
    logits = flat @ router_w.T + router_b
    probs = jax.nn.softmax(logits.astype(jnp.float32), axis=-1)
    pmax = jnp.max(probs, axis=-1, keepdims=True)
    eidx2d = jnp.argmax(probs, axis=-1, keepdims=True).astype(jnp.int32)
    gate2d = pmax / (pmax + 1e-9)
    stok2d, gslot2d, sgat2d, aux2d = _router_call(probs, eidx2d, gate2d)
    stok = stok2d.reshape(NSLOT)
    sgat = sgat2d.reshape(N)
    gslot = gslot2d.reshape(E, 1, CAPP)
    xd = _make_sc_gather(NSLOT, DIM)(flat, stok)
    yd = _expert_call(xd, w1, b1.reshape(E, 1, HIDDEN), w2,
                      b2.reshape(E, 1, DIM), gslot)
    out = _make_sc_gather(N, DIM)(yd, sgat)
    return out.reshape(b, n, d), aux2d[0, 0]
